# baseline (speedup 1.0000x reference)
import functools

import jax
import jax.numpy as jnp
from jax import lax
from jax.experimental import pallas as pl
from jax.experimental.pallas import tpu as pltpu
from jax.experimental.pallas import tpu_sc as plsc


@functools.lru_cache(maxsize=None)
def _make_gather(V, D, B, S):
    info = plsc.get_sparse_core_info()
    NC, NS = info.num_cores, info.num_subcores
    NW = NC * NS
    rows_w = B // NW

    mesh = plsc.VectorSubcoreMesh(core_axis_name="c", subcore_axis_name="s")

    @functools.partial(
        pl.kernel,
        mesh=mesh,
        out_type=jax.ShapeDtypeStruct((B, S, D), jnp.float32),
        compiler_params=pltpu.CompilerParams(use_tc_tiling_on_sc=True),
        scratch_types=[
            pltpu.VMEM((rows_w, S), jnp.int32),
            pltpu.VMEM((S, 128), jnp.float32),
            pltpu.VMEM((S, D), jnp.float32),
            pltpu.SemaphoreType.DMA,
            pltpu.SemaphoreType.DMA,
        ],
    )
    def k(table_hbm, idx_hbm, out_hbm, idx_v, gbuf, cbuf, sem_g, sem_w):
        wid = lax.axis_index("s") * NC + lax.axis_index("c")
        base = wid * rows_w

        pltpu.sync_copy(idx_hbm.at[pl.ds(base, rows_w)], idx_v)

        def body(s, carry):
            pltpu.async_copy(
                table_hbm.at[idx_v.at[s, pl.ds(0, 128)]],
                gbuf.at[pl.ds(0, 128), :],
                sem_g,
            )
            pltpu.async_copy(
                table_hbm.at[idx_v.at[s, pl.ds(128, 72)]],
                gbuf.at[pl.ds(128, 72), :],
                sem_g,
            )
            pltpu.make_async_copy(table_hbm.at[pl.ds(0, S)], gbuf, sem_g).wait()

            def copy_row(r, c):
                for j in range(D // 16):
                    cbuf[r, pl.ds(j * 16, 16)] = gbuf[r, pl.ds(j * 16, 16)]
                return c

            lax.fori_loop(0, S, copy_row, 0)
            pltpu.async_copy(cbuf, out_hbm.at[base + s], sem_w)
            pltpu.make_async_copy(cbuf, out_hbm.at[0], sem_w).wait()
            return carry

        lax.fori_loop(0, rows_w, body, 0)

    return k


def kernel(x, table):
    B, S = x.shape
    V, D = table.shape
    table_p = jnp.pad(table, ((0, 0), (0, 128 - D)))
    return _make_gather(V, D, B, S)(table_p, x.astype(jnp.int32))


# speedup vs baseline: 1.1373x; 1.1373x over previous
import functools

import jax
import jax.numpy as jnp
from jax import lax
from jax.experimental import pallas as pl
from jax.experimental.pallas import tpu as pltpu
from jax.experimental.pallas import tpu_sc as plsc

C0, C1 = 128, 72


@functools.lru_cache(maxsize=None)
def _make_gather(V, D, B, S):
    info = plsc.get_sparse_core_info()
    NC, NS = info.num_cores, info.num_subcores
    NW = NC * NS
    rows_w = B // NW
    assert B % NW == 0 and S == C0 + C1

    mesh = plsc.VectorSubcoreMesh(core_axis_name="c", subcore_axis_name="s")

    @functools.partial(
        pl.kernel,
        mesh=mesh,
        out_type=jax.ShapeDtypeStruct((B, S, D), jnp.float32),
        compiler_params=pltpu.CompilerParams(use_tc_tiling_on_sc=True),
        scratch_types=[
            pltpu.VMEM((rows_w, S), jnp.int32),
            pltpu.VMEM((2, C0, 128), jnp.float32),
            pltpu.VMEM((2, C0, D), jnp.float32),
            pltpu.SemaphoreType.DMA,
            pltpu.SemaphoreType.DMA,
        ],
    )
    def k(table_hbm, idx_hbm, out_hbm, idx_v, gbufs, cbufs, sem_g, sem_w):
        wid = lax.axis_index("s") * NC + lax.axis_index("c")
        base = wid * rows_w

        pltpu.sync_copy(idx_hbm.at[pl.ds(base, rows_w)], idx_v)

        def fire(r, off, n, gbuf):
            pltpu.async_copy(
                table_hbm.at[idx_v.at[r, pl.ds(off, n)]],
                gbuf.at[pl.ds(0, n), :],
                sem_g,
            )

        def drain_gather(n, gbuf):
            pltpu.make_async_copy(
                table_hbm.at[pl.ds(0, n)], gbuf.at[pl.ds(0, n), :], sem_g
            ).wait()

        def compact(n, gbuf, cbuf):
            def row(r, c):
                for j in range(D // 16):
                    cbuf[r, pl.ds(j * 16, 16)] = gbuf[r, pl.ds(j * 16, 16)]
                return c

            lax.fori_loop(0, n, row, 0)

        def fire_wb(r, off, n, cbuf):
            pltpu.async_copy(
                cbuf.at[pl.ds(0, n), :], out_hbm.at[base + r, pl.ds(off, n)], sem_w
            )

        def drain_wb(n, cbuf):
            pltpu.make_async_copy(
                cbuf.at[pl.ds(0, n), :], out_hbm.at[0, pl.ds(0, n)], sem_w
            ).wait()

        g0, g1 = gbufs.at[0], gbufs.at[1]
        c0, c1 = cbufs.at[0], cbufs.at[1]

        fire(0, 0, C0, g0)

        def body(r, carry):
            fire(r, C0, C1, g1)
            drain_gather(C0, g0)

            @pl.when(r > 0)
            def _():
                drain_wb(C0, c0)

            compact(C0, g0, c0)
            fire_wb(r, 0, C0, c0)

            @pl.when(r < rows_w - 1)
            def _():
                fire(r + 1, 0, C0, g0)

            drain_gather(C1, g1)

            @pl.when(r > 0)
            def _():
                drain_wb(C1, c1)

            compact(C1, g1, c1)
            fire_wb(r, C0, C1, c1)
            return carry

        lax.fori_loop(0, rows_w, body, 0)
        drain_wb(C0, c0)
        drain_wb(C1, c1)

    return k


def kernel(x, table):
    B, S = x.shape
    V, D = table.shape
    table_p = jnp.pad(table, ((0, 0), (0, 128 - D)))
    return _make_gather(V, D, B, S)(table_p, x.astype(jnp.int32))


# speedup vs baseline: 1.1443x; 1.0062x over previous
import functools

import jax
import jax.numpy as jnp
from jax import lax
from jax.experimental import pallas as pl
from jax.experimental.pallas import tpu as pltpu
from jax.experimental.pallas import tpu_sc as plsc

C0, C1 = 128, 72


@functools.lru_cache(maxsize=None)
def _make_gather(V, D, B, S):
    info = plsc.get_sparse_core_info()
    NC, NS = info.num_cores, info.num_subcores
    NW = NC * NS
    rows_w = B // NW
    assert B % NW == 0 and S == C0 + C1

    mesh = plsc.VectorSubcoreMesh(core_axis_name="c", subcore_axis_name="s")

    @functools.partial(
        pl.kernel,
        mesh=mesh,
        out_type=jax.ShapeDtypeStruct((B, S, D), jnp.float32),
        compiler_params=pltpu.CompilerParams(use_tc_tiling_on_sc=True),
        scratch_types=[
            pltpu.VMEM((rows_w, S), jnp.int32),
            pltpu.VMEM((2, C0, 128), jnp.float32),
            pltpu.VMEM((2, C0, D), jnp.float32),
            pltpu.SemaphoreType.DMA,
            pltpu.SemaphoreType.DMA,
        ],
    )
    def k(table_hbm, idx_hbm, out_hbm, idx_v, gbufs, cbufs, sem_g, sem_w):
        wid = lax.axis_index("s") * NC + lax.axis_index("c")
        base = wid * rows_w

        pltpu.sync_copy(idx_hbm.at[pl.ds(base, rows_w)], idx_v)

        def fire(r, off, n, gbuf):
            pltpu.async_copy(
                table_hbm.at[idx_v.at[r, pl.ds(off, n)]],
                gbuf.at[pl.ds(0, n), :],
                sem_g,
            )

        def drain_gather(n, gbuf):
            pltpu.make_async_copy(
                table_hbm.at[pl.ds(0, n)], gbuf.at[pl.ds(0, n), :], sem_g
            ).wait()

        def compact(n, gbuf, cbuf):
            assert n % 4 == 0

            def rows4(i, c):
                r = i * 4
                for dr in range(4):
                    for j in range(D // 16):
                        cbuf[r + dr, pl.ds(j * 16, 16)] = gbuf[
                            r + dr, pl.ds(j * 16, 16)
                        ]
                return c

            lax.fori_loop(0, n // 4, rows4, 0)

        def fire_wb(r, off, n, cbuf):
            pltpu.async_copy(
                cbuf.at[pl.ds(0, n), :], out_hbm.at[base + r, pl.ds(off, n)], sem_w
            )

        def drain_wb(n, cbuf):
            pltpu.make_async_copy(
                cbuf.at[pl.ds(0, n), :], out_hbm.at[0, pl.ds(0, n)], sem_w
            ).wait()

        g0, g1 = gbufs.at[0], gbufs.at[1]
        c0, c1 = cbufs.at[0], cbufs.at[1]

        fire(0, 0, C0, g0)

        def body(r, carry):
            fire(r, C0, C1, g1)
            drain_gather(C0, g0)

            @pl.when(r > 0)
            def _():
                drain_wb(C0, c0)

            compact(C0, g0, c0)
            fire_wb(r, 0, C0, c0)

            @pl.when(r < rows_w - 1)
            def _():
                fire(r + 1, 0, C0, g0)

            drain_gather(C1, g1)

            @pl.when(r > 0)
            def _():
                drain_wb(C1, c1)

            compact(C1, g1, c1)
            fire_wb(r, C0, C1, c1)
            return carry

        lax.fori_loop(0, rows_w, body, 0)
        drain_wb(C0, c0)
        drain_wb(C1, c1)

    return k


def kernel(x, table):
    B, S = x.shape
    V, D = table.shape
    table_p = jnp.pad(table, ((0, 0), (0, 128 - D)))
    return _make_gather(V, D, B, S)(table_p, x.astype(jnp.int32))


# speedup vs baseline: 1.2603x; 1.1013x over previous
import functools

import jax
import jax.numpy as jnp
from jax import lax
from jax.experimental import pallas as pl
from jax.experimental.pallas import tpu as pltpu
from jax.experimental.pallas import tpu_sc as plsc

C0, C1 = 128, 72


@functools.lru_cache(maxsize=None)
def _make_gather(V, D, B, S):
    info = plsc.get_sparse_core_info()
    NC, NS = info.num_cores, info.num_subcores
    NW = NC * NS
    rows_w = B // NW
    assert B % NW == 0 and S == C0 + C1

    mesh = plsc.VectorSubcoreMesh(core_axis_name="c", subcore_axis_name="s")

    @functools.partial(
        pl.kernel,
        mesh=mesh,
        out_type=jax.ShapeDtypeStruct((B * S, D), jnp.float32),
        compiler_params=pltpu.CompilerParams(use_tc_tiling_on_sc=True),
        scratch_types=[
            pltpu.VMEM((rows_w, S), jnp.int32),
            pltpu.VMEM((2, C0, 128), jnp.float32),
            pltpu.VMEM((2, C0, D), jnp.float32),
            pltpu.SemaphoreType.DMA,
            pltpu.SemaphoreType.DMA,
        ],
    )
    def k(table_hbm, idx_hbm, out_hbm, idx_v, gbufs, cbufs, sem_g, sem_w):
        wid = lax.axis_index("s") * NC + lax.axis_index("c")
        base = wid * rows_w

        pltpu.sync_copy(idx_hbm.at[pl.ds(base, rows_w)], idx_v)

        def fire(r, off, n, gbuf):
            pltpu.async_copy(
                table_hbm.at[idx_v.at[r, pl.ds(off, n)]],
                gbuf.at[pl.ds(0, n), :],
                sem_g,
            )

        def drain_gather(n, gbuf):
            pltpu.make_async_copy(
                table_hbm.at[pl.ds(0, n)], gbuf.at[pl.ds(0, n), :], sem_g
            ).wait()

        def compact(n, gbuf, cbuf):
            assert n % 4 == 0

            def rows4(i, c):
                r = i * 4
                for dr in range(4):
                    for j in range(D // 16):
                        cbuf[r + dr, pl.ds(j * 16, 16)] = gbuf[
                            r + dr, pl.ds(j * 16, 16)
                        ]
                return c

            lax.fori_loop(0, n // 4, rows4, 0)

        def fire_wb(r, off, n, cbuf):
            pltpu.async_copy(
                cbuf.at[pl.ds(0, n), :],
                out_hbm.at[pl.ds((base + r) * S + off, n)],
                sem_w,
            )

        def drain_wb(n, cbuf):
            pltpu.make_async_copy(
                cbuf.at[pl.ds(0, n), :], out_hbm.at[pl.ds(0, n)], sem_w
            ).wait()

        g0, g1 = gbufs.at[0], gbufs.at[1]
        c0, c1 = cbufs.at[0], cbufs.at[1]

        fire(0, 0, C0, g0)

        def body(r, carry):
            fire(r, C0, C1, g1)
            drain_gather(C0, g0)

            @pl.when(r > 0)
            def _():
                drain_wb(C0, c0)

            compact(C0, g0, c0)
            fire_wb(r, 0, C0, c0)

            @pl.when(r < rows_w - 1)
            def _():
                fire(r + 1, 0, C0, g0)

            drain_gather(C1, g1)

            @pl.when(r > 0)
            def _():
                drain_wb(C1, c1)

            compact(C1, g1, c1)
            fire_wb(r, C0, C1, c1)
            return carry

        lax.fori_loop(0, rows_w, body, 0)
        drain_wb(C0, c0)
        drain_wb(C1, c1)

    return k


def kernel(x, table):
    B, S = x.shape
    V, D = table.shape
    table_p = jnp.pad(table, ((0, 0), (0, 128 - D)))
    out = _make_gather(V, D, B, S)(table_p, x.astype(jnp.int32))
    return out.reshape(B, S, D)


# speedup vs baseline: 1.2609x; 1.0005x over previous
import functools

import jax
import jax.numpy as jnp
from jax import lax
from jax.experimental import pallas as pl
from jax.experimental.pallas import tpu as pltpu
from jax.experimental.pallas import tpu_sc as plsc

C0, C1 = 128, 72


@functools.lru_cache(maxsize=None)
def _make_gather(V, D, B, S):
    info = plsc.get_sparse_core_info()
    NC, NS = info.num_cores, info.num_subcores
    NW = NC * NS
    rows_w = B // NW
    assert B % NW == 0 and S == C0 + C1

    mesh = plsc.VectorSubcoreMesh(core_axis_name="c", subcore_axis_name="s")

    @functools.partial(
        pl.kernel,
        mesh=mesh,
        out_type=jax.ShapeDtypeStruct((B * S, D), jnp.float32),
        compiler_params=pltpu.CompilerParams(use_tc_tiling_on_sc=True),
        scratch_types=[
            pltpu.VMEM((rows_w, S), jnp.int32),
            pltpu.VMEM((2, C0, 128), jnp.float32),
            pltpu.VMEM((2, C0, D), jnp.float32),
            pltpu.SemaphoreType.DMA,
            pltpu.SemaphoreType.DMA,
        ],
    )
    def k(table_hbm, idx_hbm, out_hbm, idx_v, gbufs, cbufs, sem_g, sem_w):
        wid = lax.axis_index("s") * NC + lax.axis_index("c")
        base = wid * rows_w

        pltpu.sync_copy(idx_hbm.at[pl.ds(base, rows_w)], idx_v)

        def fire(r, off, n, gbuf):
            h = n // 2 // 8 * 8
            for o, m in ((0, h), (h, n - h)):
                pltpu.async_copy(
                    table_hbm.at[idx_v.at[r, pl.ds(off + o, m)]],
                    gbuf.at[pl.ds(o, m), :],
                    sem_g,
                )

        def drain_gather(n, gbuf):
            pltpu.make_async_copy(
                table_hbm.at[pl.ds(0, n)], gbuf.at[pl.ds(0, n), :], sem_g
            ).wait()

        def compact(n, gbuf, cbuf):
            assert n % 4 == 0

            def rows4(i, c):
                r = i * 4
                for dr in range(4):
                    for j in range(D // 16):
                        cbuf[r + dr, pl.ds(j * 16, 16)] = gbuf[
                            r + dr, pl.ds(j * 16, 16)
                        ]
                return c

            lax.fori_loop(0, n // 4, rows4, 0)

        def fire_wb(r, off, n, cbuf):
            pltpu.async_copy(
                cbuf.at[pl.ds(0, n), :],
                out_hbm.at[pl.ds((base + r) * S + off, n)],
                sem_w,
            )

        def drain_wb(n, cbuf):
            pltpu.make_async_copy(
                cbuf.at[pl.ds(0, n), :], out_hbm.at[pl.ds(0, n)], sem_w
            ).wait()

        g0, g1 = gbufs.at[0], gbufs.at[1]
        c0, c1 = cbufs.at[0], cbufs.at[1]

        fire(0, 0, C0, g0)

        def body(r, carry):
            fire(r, C0, C1, g1)
            drain_gather(C0, g0)

            @pl.when(r > 0)
            def _():
                drain_wb(C0, c0)

            compact(C0, g0, c0)
            fire_wb(r, 0, C0, c0)

            @pl.when(r < rows_w - 1)
            def _():
                fire(r + 1, 0, C0, g0)

            drain_gather(C1, g1)

            @pl.when(r > 0)
            def _():
                drain_wb(C1, c1)

            compact(C1, g1, c1)
            fire_wb(r, C0, C1, c1)
            return carry

        lax.fori_loop(0, rows_w, body, 0)
        drain_wb(C0, c0)
        drain_wb(C1, c1)

    return k


def kernel(x, table):
    B, S = x.shape
    V, D = table.shape
    table_p = jnp.pad(table, ((0, 0), (0, 128 - D)))
    out = _make_gather(V, D, B, S)(table_p, x.astype(jnp.int32))
    return out.reshape(B, S, D)
